# baseline (device time: 15392 ns/iter reference)
import jax
import jax.numpy as jnp
from jax import lax
from jax.experimental import pallas as pl
from jax.experimental.pallas import tpu as pltpu

CHUNKS = (32, 96, 96, 32)
C = len(CHUNKS)
OFFS = tuple(sum(CHUNKS[:i]) for i in range(C))


def kernel(dy, W):
    m, k = dy.shape
    d, _ = W.shape
    mb = m // 2
    d2 = d // 2

    dy = pltpu.with_memory_space_constraint(dy, pltpu.MemorySpace.HBM)
    W = pltpu.with_memory_space_constraint(W, pltpu.MemorySpace.HBM)

    def body(
        dy_ref,
        w_ref,
        out_ref,
        wv,
        dyv,
        pf32,
        pbuf,
        xrecv,
        ybuf,
        yrecv,
        wa_sem,
        wb_sem,
        dy_sem,
        x_send_sems,
        x_recv_sems,
        y_send_sems,
        y_recv_sems,
    ):
        my_x = lax.axis_index("x")
        my_y = lax.axis_index("y")
        base = my_y * mb
        peer_base = (1 - my_y) * mb

        dy_copy = pltpu.make_async_copy(
            dy_ref.at[pl.ds(base, mb)], dyv, dy_sem
        )
        dy_copy.start()
        wa_copy = pltpu.make_async_copy(
            w_ref.at[pl.ds(0, d2)], wv.at[pl.ds(0, d2)], wa_sem
        )
        wa_copy.start()
        wb_copy = pltpu.make_async_copy(
            w_ref.at[pl.ds(d2, d2)], wv.at[pl.ds(d2, d2)], wb_sem
        )
        wb_copy.start()

        dy_copy.wait()
        wa_copy.wait()
        pf32[:, pl.ds(0, d2)] = lax.dot_general(
            dyv[:, :],
            wv[pl.ds(0, d2), :],
            dimension_numbers=(((1,), (1,)), ((), ())),
            preferred_element_type=jnp.float32,
        )
        wb_copy.wait()
        pf32[:, pl.ds(d2, d2)] = lax.dot_general(
            dyv[:, :],
            wv[pl.ds(d2, d2), :],
            dimension_numbers=(((1,), (1,)), ((), ())),
            preferred_element_type=jnp.float32,
        )
        pbuf[:, :] = pf32[:, :].astype(jnp.bfloat16)

        barrier_sem = pltpu.get_barrier_semaphore()
        pl.semaphore_signal(
            barrier_sem, inc=1, device_id=(1 - my_x, my_y),
            device_id_type=pl.DeviceIdType.MESH,
        )
        pl.semaphore_signal(
            barrier_sem, inc=1, device_id=(my_x, 1 - my_y),
            device_id_type=pl.DeviceIdType.MESH,
        )
        pl.semaphore_wait(barrier_sem, 2)

        x_rdmas = []
        for c in range(C):
            sl = pl.ds(OFFS[c], CHUNKS[c])
            r = pltpu.make_async_remote_copy(
                src_ref=pbuf.at[sl],
                dst_ref=xrecv.at[sl],
                send_sem=x_send_sems.at[c],
                recv_sem=x_recv_sems.at[c],
                device_id=(1 - my_x, my_y),
                device_id_type=pl.DeviceIdType.MESH,
            )
            r.start()
            x_rdmas.append(r)

        y_rdmas = []
        for c in range(C):
            sl = pl.ds(OFFS[c], CHUNKS[c])
            x_rdmas[c].wait_recv()
            sum_bf = pbuf[sl, :] + xrecv[sl, :]
            ybuf[sl, :] = sum_bf
            out_ref[pl.ds(base + OFFS[c], CHUNKS[c]), :] = sum_bf.astype(
                jnp.float32
            )
            s = pltpu.make_async_remote_copy(
                src_ref=ybuf.at[sl],
                dst_ref=yrecv.at[sl],
                send_sem=y_send_sems.at[c],
                recv_sem=y_recv_sems.at[c],
                device_id=(my_x, 1 - my_y),
                device_id_type=pl.DeviceIdType.MESH,
            )
            s.start()
            y_rdmas.append(s)

        for c in range(C):
            y_rdmas[c].wait_recv()
            out_ref[pl.ds(peer_base + OFFS[c], CHUNKS[c]), :] = yrecv[
                pl.ds(OFFS[c], CHUNKS[c]), :
            ].astype(jnp.float32)

        for c in range(C):
            x_rdmas[c].wait_send()
            y_rdmas[c].wait_send()

    return pl.pallas_call(
        body,
        out_shape=jax.ShapeDtypeStruct((m, d), jnp.float32),
        in_specs=[
            pl.BlockSpec(memory_space=pltpu.MemorySpace.HBM),
            pl.BlockSpec(memory_space=pltpu.MemorySpace.HBM),
        ],
        out_specs=pl.BlockSpec(memory_space=pltpu.VMEM),
        scratch_shapes=[
            pltpu.VMEM((d, k), jnp.float32),
            pltpu.VMEM((mb, k), jnp.float32),
            pltpu.VMEM((mb, d), jnp.float32),
            pltpu.VMEM((mb, d), jnp.bfloat16),
            pltpu.VMEM((mb, d), jnp.bfloat16),
            pltpu.VMEM((mb, d), jnp.bfloat16),
            pltpu.VMEM((mb, d), jnp.bfloat16),
            pltpu.SemaphoreType.DMA,
            pltpu.SemaphoreType.DMA,
            pltpu.SemaphoreType.DMA,
            pltpu.SemaphoreType.DMA((C,)),
            pltpu.SemaphoreType.DMA((C,)),
            pltpu.SemaphoreType.DMA((C,)),
            pltpu.SemaphoreType.DMA((C,)),
        ],
        compiler_params=pltpu.CompilerParams(collective_id=0),
    )(dy, W)


# device time: 14420 ns/iter; 1.0674x vs baseline; 1.0674x over previous
import jax
import jax.numpy as jnp
from jax import lax
from jax.experimental import pallas as pl
from jax.experimental.pallas import tpu as pltpu

C = 4


def kernel(dy, W):
    m, k = dy.shape
    d, _ = W.shape
    mb = m // 2
    rc = mb // C

    dy = pltpu.with_memory_space_constraint(dy, pltpu.MemorySpace.HBM)
    W = pltpu.with_memory_space_constraint(W, pltpu.MemorySpace.HBM)

    def body(
        dy_ref,
        w_ref,
        out_ref,
        wv,
        dyv,
        pbuf,
        xbuf,
        xrecv,
        ybuf,
        yrecv,
        w_sem,
        dy_sem,
        x_send_sems,
        x_recv_sems,
        y_send_sems,
        y_recv_sems,
    ):
        my_x = lax.axis_index("x")
        my_y = lax.axis_index("y")
        base = my_y * mb
        peer_base = (1 - my_y) * mb

        w_copy = pltpu.make_async_copy(w_ref, wv, w_sem)
        w_copy.start()
        dy_copy = pltpu.make_async_copy(
            dy_ref.at[pl.ds(base, mb)], dyv, dy_sem
        )
        dy_copy.start()

        w_copy.wait()
        dy_copy.wait()
        pbuf[:, :] = lax.dot_general(
            dyv[:, :],
            wv[:, :],
            dimension_numbers=(((1,), (1,)), ((), ())),
            preferred_element_type=jnp.float32,
        )
        for c in range(C):
            xbuf[c, :, :] = pbuf[pl.ds(c * rc, rc), :].astype(jnp.bfloat16)

        barrier_sem = pltpu.get_barrier_semaphore()
        pl.semaphore_signal(
            barrier_sem, inc=1, device_id=(1 - my_x, my_y),
            device_id_type=pl.DeviceIdType.MESH,
        )
        pl.semaphore_signal(
            barrier_sem, inc=1, device_id=(my_x, 1 - my_y),
            device_id_type=pl.DeviceIdType.MESH,
        )
        pl.semaphore_wait(barrier_sem, 2)

        x_rdmas = []
        for c in range(C):
            r = pltpu.make_async_remote_copy(
                src_ref=xbuf.at[c],
                dst_ref=xrecv.at[c],
                send_sem=x_send_sems.at[c],
                recv_sem=x_recv_sems.at[c],
                device_id=(1 - my_x, my_y),
                device_id_type=pl.DeviceIdType.MESH,
            )
            r.start()
            x_rdmas.append(r)

        y_rdmas = []
        for c in range(C):
            x_rdmas[c].wait_recv()
            red = pbuf[pl.ds(c * rc, rc), :] + xrecv[c].astype(jnp.float32)
            out_ref[pl.ds(base + c * rc, rc), :] = red
            ybuf[c, :, :] = red.astype(jnp.bfloat16)
            s = pltpu.make_async_remote_copy(
                src_ref=ybuf.at[c],
                dst_ref=yrecv.at[c],
                send_sem=y_send_sems.at[c],
                recv_sem=y_recv_sems.at[c],
                device_id=(my_x, 1 - my_y),
                device_id_type=pl.DeviceIdType.MESH,
            )
            s.start()
            y_rdmas.append(s)

        for c in range(C):
            y_rdmas[c].wait_recv()
            out_ref[pl.ds(peer_base + c * rc, rc), :] = yrecv[c].astype(
                jnp.float32
            )

        for c in range(C):
            x_rdmas[c].wait_send()
            y_rdmas[c].wait_send()

    return pl.pallas_call(
        body,
        out_shape=jax.ShapeDtypeStruct((m, d), jnp.float32),
        in_specs=[
            pl.BlockSpec(memory_space=pltpu.MemorySpace.HBM),
            pl.BlockSpec(memory_space=pltpu.MemorySpace.HBM),
        ],
        out_specs=pl.BlockSpec(memory_space=pltpu.VMEM),
        scratch_shapes=[
            pltpu.VMEM((d, k), jnp.float32),
            pltpu.VMEM((mb, k), jnp.float32),
            pltpu.VMEM((mb, d), jnp.float32),
            pltpu.VMEM((C, rc, d), jnp.bfloat16),
            pltpu.VMEM((C, rc, d), jnp.bfloat16),
            pltpu.VMEM((C, rc, d), jnp.bfloat16),
            pltpu.VMEM((C, rc, d), jnp.bfloat16),
            pltpu.SemaphoreType.DMA,
            pltpu.SemaphoreType.DMA,
            pltpu.SemaphoreType.DMA((C,)),
            pltpu.SemaphoreType.DMA((C,)),
            pltpu.SemaphoreType.DMA((C,)),
            pltpu.SemaphoreType.DMA((C,)),
        ],
        compiler_params=pltpu.CompilerParams(collective_id=0),
    )(dy, W)


# device time: 14206 ns/iter; 1.0835x vs baseline; 1.0151x over previous
import jax
import jax.numpy as jnp
from jax import lax
from jax.experimental import pallas as pl
from jax.experimental.pallas import tpu as pltpu

C = 8


def kernel(dy, W):
    m, k = dy.shape
    d, _ = W.shape
    mb = m // 2
    rc = mb // C

    dy = pltpu.with_memory_space_constraint(dy, pltpu.MemorySpace.HBM)
    W = pltpu.with_memory_space_constraint(W, pltpu.MemorySpace.HBM)

    def body(
        dy_ref,
        w_ref,
        out_ref,
        wv,
        dyv,
        pbuf,
        xbuf,
        xrecv,
        ybuf,
        yrecv,
        w_sem,
        dy_sem,
        x_send_sems,
        x_recv_sems,
        y_send_sems,
        y_recv_sems,
    ):
        my_x = lax.axis_index("x")
        my_y = lax.axis_index("y")
        base = my_y * mb
        peer_base = (1 - my_y) * mb

        w_copy = pltpu.make_async_copy(w_ref, wv, w_sem)
        w_copy.start()
        dy_copy = pltpu.make_async_copy(
            dy_ref.at[pl.ds(base, mb)], dyv, dy_sem
        )
        dy_copy.start()

        w_copy.wait()
        dy_copy.wait()
        pbuf[:, :] = lax.dot_general(
            dyv[:, :],
            wv[:, :],
            dimension_numbers=(((1,), (1,)), ((), ())),
            preferred_element_type=jnp.float32,
        )
        for c in range(C):
            xbuf[c, :, :] = pbuf[pl.ds(c * rc, rc), :].astype(jnp.bfloat16)

        barrier_sem = pltpu.get_barrier_semaphore()
        pl.semaphore_signal(
            barrier_sem, inc=1, device_id=(1 - my_x, my_y),
            device_id_type=pl.DeviceIdType.MESH,
        )
        pl.semaphore_signal(
            barrier_sem, inc=1, device_id=(my_x, 1 - my_y),
            device_id_type=pl.DeviceIdType.MESH,
        )
        pl.semaphore_wait(barrier_sem, 2)

        x_rdmas = []
        for c in range(C):
            r = pltpu.make_async_remote_copy(
                src_ref=xbuf.at[c],
                dst_ref=xrecv.at[c],
                send_sem=x_send_sems.at[c],
                recv_sem=x_recv_sems.at[c],
                device_id=(1 - my_x, my_y),
                device_id_type=pl.DeviceIdType.MESH,
            )
            r.start()
            x_rdmas.append(r)

        y_rdmas = []
        for c in range(C):
            x_rdmas[c].wait_recv()
            red = pbuf[pl.ds(c * rc, rc), :] + xrecv[c].astype(jnp.float32)
            out_ref[pl.ds(base + c * rc, rc), :] = red
            ybuf[c, :, :] = red.astype(jnp.bfloat16)
            s = pltpu.make_async_remote_copy(
                src_ref=ybuf.at[c],
                dst_ref=yrecv.at[c],
                send_sem=y_send_sems.at[c],
                recv_sem=y_recv_sems.at[c],
                device_id=(my_x, 1 - my_y),
                device_id_type=pl.DeviceIdType.MESH,
            )
            s.start()
            y_rdmas.append(s)

        for c in range(C):
            y_rdmas[c].wait_recv()
            out_ref[pl.ds(peer_base + c * rc, rc), :] = yrecv[c].astype(
                jnp.float32
            )

        for c in range(C):
            x_rdmas[c].wait_send()
            y_rdmas[c].wait_send()

    return pl.pallas_call(
        body,
        out_shape=jax.ShapeDtypeStruct((m, d), jnp.float32),
        in_specs=[
            pl.BlockSpec(memory_space=pltpu.MemorySpace.HBM),
            pl.BlockSpec(memory_space=pltpu.MemorySpace.HBM),
        ],
        out_specs=pl.BlockSpec(memory_space=pltpu.VMEM),
        scratch_shapes=[
            pltpu.VMEM((d, k), jnp.float32),
            pltpu.VMEM((mb, k), jnp.float32),
            pltpu.VMEM((mb, d), jnp.float32),
            pltpu.VMEM((C, rc, d), jnp.bfloat16),
            pltpu.VMEM((C, rc, d), jnp.bfloat16),
            pltpu.VMEM((C, rc, d), jnp.bfloat16),
            pltpu.VMEM((C, rc, d), jnp.bfloat16),
            pltpu.SemaphoreType.DMA,
            pltpu.SemaphoreType.DMA,
            pltpu.SemaphoreType.DMA((C,)),
            pltpu.SemaphoreType.DMA((C,)),
            pltpu.SemaphoreType.DMA((C,)),
            pltpu.SemaphoreType.DMA((C,)),
        ],
        compiler_params=pltpu.CompilerParams(collective_id=0),
    )(dy, W)
